# Initial kernel scaffold; baseline (speedup 1.0000x reference)
import jax
import jax.numpy as jnp
from jax.experimental import pallas as pl


def kernel(x, router_scale, per_expert_scale, router_w, gate_up_proj, down_proj):
    raise NotImplementedError("write your pallas kernel here")


# speedup vs baseline: 3.4169x; 3.4169x over previous
import functools

import jax
import jax.numpy as jnp
from jax import lax
from jax.experimental import pallas as pl
from jax.experimental.pallas import tpu as pltpu

N_TOK = 2048
HID = 768
EDIM = 512
NE = 64
BLK = 64
RPAD = 8192
NBLK = RPAD // BLK


def _gelu_tanh(x):
    return 0.5 * x * (1.0 + jnp.tanh(jnp.sqrt(2.0 / jnp.pi) * (x + 0.044715 * x ** 3)))


def _router_plan_body(x_ref, rs_ref, pes_ref, rw_ref, post_ref, wt_ref, b2e_ref):
    x = x_ref[...]
    xr = x * lax.rsqrt(jnp.mean(x * x, axis=-1, keepdims=True) + 1e-6)
    ri = xr * rs_ref[...] * (HID ** -0.5)
    logits = lax.dot_general(ri, rw_ref[...], (((1,), (1,)), ((), ())),
                             preferred_element_type=jnp.float32)
    m = jnp.max(logits, axis=-1, keepdims=True)
    p = jnp.exp(logits - m)
    probs = p / jnp.sum(p, axis=-1, keepdims=True)

    e_iota = lax.broadcasted_iota(jnp.int32, (N_TOK, NE), 1)
    l1 = jnp.max(logits, axis=-1, keepdims=True)
    i1 = jnp.min(jnp.where(logits == l1, e_iota, NE), axis=-1, keepdims=True)
    masked = jnp.where(e_iota == i1, -jnp.inf, logits)
    l2 = jnp.max(masked, axis=-1, keepdims=True)
    i2 = jnp.min(jnp.where(masked == l2, e_iota, NE), axis=-1, keepdims=True)

    oh1 = (e_iota == i1).astype(jnp.float32)
    oh2 = (e_iota == i2).astype(jnp.float32)
    pes = pes_ref[...]
    p1 = jnp.sum(oh1 * probs, axis=-1, keepdims=True)
    p2 = jnp.sum(oh2 * probs, axis=-1, keepdims=True)
    s = p1 + p2
    w1 = p1 / s * jnp.sum(oh1 * pes, axis=-1, keepdims=True)
    w2 = p2 / s * jnp.sum(oh2 * pes, axis=-1, keepdims=True)

    M = jnp.concatenate([oh1, oh2], axis=0)
    c = M
    sh = 1
    while sh < 2 * N_TOK:
        c = c + jnp.concatenate(
            [jnp.zeros((sh, NE), jnp.float32), c[:2 * N_TOK - sh]], axis=0)
        sh *= 2
    rank = jnp.sum(M * (c - M), axis=-1, keepdims=True)

    counts = jnp.sum(M, axis=0, keepdims=True)
    bpe = jnp.floor((counts + (BLK - 1)) * (1.0 / BLK))
    ut = (lax.broadcasted_iota(jnp.int32, (NE, NE), 0)
          <= lax.broadcasted_iota(jnp.int32, (NE, NE), 1)).astype(jnp.float32)
    cumb = lax.dot_general(bpe, ut, (((1,), (0,)), ((), ())),
                           preferred_element_type=jnp.float32)
    row_off = (cumb - bpe) * float(BLK)
    pos = jnp.sum(M * row_off, axis=-1, keepdims=True) + rank
    post_ref[...] = pos.astype(jnp.int32).reshape(2, N_TOK)
    wt_ref[...] = jnp.concatenate([w1, w2], axis=0).reshape(2, N_TOK)

    b_iota = lax.broadcasted_iota(jnp.int32, (NBLK, NE), 0)
    b2e = jnp.sum((cumb.astype(jnp.int32) <= b_iota).astype(jnp.int32),
                  axis=-1, keepdims=True)
    last_used = jnp.max(jnp.where(counts > 0,
                                  lax.broadcasted_iota(jnp.int32, (1, NE), 1),
                                  0), axis=-1, keepdims=True)
    b2e = jnp.minimum(b2e, last_used)
    b2e_ref[...] = b2e.reshape(1, NBLK)


def _router_plan(x, router_scale, per_expert_scale, router_w):
    return pl.pallas_call(
        _router_plan_body,
        out_shape=[
            jax.ShapeDtypeStruct((2, N_TOK), jnp.int32),
            jax.ShapeDtypeStruct((2, N_TOK), jnp.float32),
            jax.ShapeDtypeStruct((1, NBLK), jnp.int32),
        ],
    )(x, router_scale.reshape(1, HID), per_expert_scale.reshape(1, NE),
      router_w)


def _mlp_body(b2e_ref, xs_ref, gup_ref, down_ref, ws_ref, out_ref):
    del b2e_ref
    xb = xs_ref[...]
    gu = lax.dot_general(xb, gup_ref[0], (((1,), (1,)), ((), ())),
                         preferred_element_type=jnp.float32)
    gate = gu[:, :EDIM]
    up = gu[:, EDIM:]
    h = _gelu_tanh(gate) * up
    o = lax.dot_general(h, down_ref[0], (((1,), (1,)), ((), ())),
                        preferred_element_type=jnp.float32)
    out_ref[...] = o * ws_ref[...]


def _grouped_mlp(xs, gate_up_proj, down_proj, w_sorted, b2e):
    grid_spec = pltpu.PrefetchScalarGridSpec(
        num_scalar_prefetch=1,
        grid=(NBLK,),
        in_specs=[
            pl.BlockSpec((BLK, HID), lambda b, s: (b, 0)),
            pl.BlockSpec((1, 2 * EDIM, HID), lambda b, s: (s[b], 0, 0)),
            pl.BlockSpec((1, HID, EDIM), lambda b, s: (s[b], 0, 0)),
            pl.BlockSpec((BLK, 1), lambda b, s: (b, 0)),
        ],
        out_specs=pl.BlockSpec((BLK, HID), lambda b, s: (b, 0)),
    )
    return pl.pallas_call(
        _mlp_body,
        grid_spec=grid_spec,
        out_shape=jax.ShapeDtypeStruct((RPAD, HID), jnp.float32),
    )(b2e, xs, gate_up_proj, down_proj, w_sorted.reshape(RPAD, 1))


def kernel(x, router_scale, per_expert_scale, router_w, gate_up_proj,
           down_proj):
    pos2, w2, b2e = _router_plan(x, router_scale, per_expert_scale, router_w)
    pos_flat = pos2.reshape(2 * N_TOK)
    w_flat = w2.reshape(2 * N_TOK)
    tok_flat = jnp.tile(jnp.arange(N_TOK, dtype=jnp.int32), 2)

    tok_sorted = jnp.zeros((RPAD,), jnp.int32).at[pos_flat].set(tok_flat)
    w_sorted = jnp.zeros((RPAD,), jnp.float32).at[pos_flat].set(w_flat)

    xs = x[tok_sorted]

    hw = _grouped_mlp(xs, gate_up_proj, down_proj, w_sorted, b2e.reshape(NBLK))

    out = hw[pos2[0]] + hw[pos2[1]]
    return out
